# baseline (speedup 1.0000x reference)
import jax
import jax.numpy as jnp
from jax.experimental import pallas as pl


def _gate_kernel(x_ref, w_ref, o0_ref, o1_ref):
    xb = x_ref[0]
    w = w_ref[...]
    logits = jax.lax.dot_general(
        xb, w, (((1,), (1,)), ((), ())),
        preferred_element_type=jnp.float32)
    hard0 = (logits[:, 0] >= logits[:, 1]).astype(jnp.float32)
    o0_ref[0, 0] = hard0
    o1_ref[0, 0] = 1.0 - hard0


def kernel(x, W):
    B, S, D = x.shape
    BLK = 1024
    n_blocks = (B * S) // BLK
    xr = x.reshape(n_blocks, BLK, D)
    out0, out1 = pl.pallas_call(
        _gate_kernel,
        grid=(n_blocks,),
        in_specs=[
            pl.BlockSpec((1, BLK, D), lambda i: (i, 0, 0)),
            pl.BlockSpec((2, D), lambda i: (0, 0)),
        ],
        out_specs=[
            pl.BlockSpec((1, 1, BLK), lambda i: (i, 0, 0)),
            pl.BlockSpec((1, 1, BLK), lambda i: (i, 0, 0)),
        ],
        out_shape=[
            jax.ShapeDtypeStruct((n_blocks, 1, BLK), jnp.float32),
            jax.ShapeDtypeStruct((n_blocks, 1, BLK), jnp.float32),
        ],
    )(xr, W)
    return out0.reshape(B, S), out1.reshape(B, S)


# speedup vs baseline: 1.2613x; 1.2613x over previous
import jax
import jax.numpy as jnp
from jax.experimental import pallas as pl


def _gate_kernel(x_ref, w_ref, o0_ref, o1_ref):
    xb = x_ref[0]
    w = w_ref[...]
    logits = jax.lax.dot_general(
        w, xb, (((1,), (1,)), ((), ())),
        preferred_element_type=jnp.float32)
    hard0 = (logits[0:1, :] >= logits[1:2, :]).astype(jnp.float32)
    o0_ref[0] = hard0
    o1_ref[0] = 1.0 - hard0


def kernel(x, W):
    B, S, D = x.shape
    BLK = 1024
    n_blocks = (B * S) // BLK
    xr = x.reshape(n_blocks, BLK, D)
    out0, out1 = pl.pallas_call(
        _gate_kernel,
        grid=(n_blocks,),
        in_specs=[
            pl.BlockSpec((1, BLK, D), lambda i: (i, 0, 0)),
            pl.BlockSpec((2, D), lambda i: (0, 0)),
        ],
        out_specs=[
            pl.BlockSpec((1, 1, BLK), lambda i: (i, 0, 0)),
            pl.BlockSpec((1, 1, BLK), lambda i: (i, 0, 0)),
        ],
        out_shape=[
            jax.ShapeDtypeStruct((n_blocks, 1, BLK), jnp.float32),
            jax.ShapeDtypeStruct((n_blocks, 1, BLK), jnp.float32),
        ],
    )(xr, W)
    return out0.reshape(B, S), out1.reshape(B, S)


# speedup vs baseline: 1.5403x; 1.2212x over previous
import jax
import jax.numpy as jnp
from jax.experimental import pallas as pl


def _gate_kernel(x_ref, w_ref, o0_ref, o1_ref):
    xb = x_ref[0]
    w = w_ref[...]
    logits = jax.lax.dot_general(
        w, xb, (((1,), (1,)), ((), ())),
        preferred_element_type=jnp.float32)
    hard0 = (logits[0:1, :] >= logits[1:2, :]).astype(jnp.float32)
    o0_ref[0] = hard0
    o1_ref[0] = 1.0 - hard0


def kernel(x, W):
    B, S, D = x.shape
    BLK = 2048
    n_blocks = (B * S) // BLK
    xr = x.reshape(n_blocks, BLK, D)
    out0, out1 = pl.pallas_call(
        _gate_kernel,
        grid=(n_blocks,),
        in_specs=[
            pl.BlockSpec((1, BLK, D), lambda i: (i, 0, 0)),
            pl.BlockSpec((2, D), lambda i: (0, 0)),
        ],
        out_specs=[
            pl.BlockSpec((1, 1, BLK), lambda i: (i, 0, 0)),
            pl.BlockSpec((1, 1, BLK), lambda i: (i, 0, 0)),
        ],
        out_shape=[
            jax.ShapeDtypeStruct((n_blocks, 1, BLK), jnp.float32),
            jax.ShapeDtypeStruct((n_blocks, 1, BLK), jnp.float32),
        ],
    )(xr, W)
    return out0.reshape(B, S), out1.reshape(B, S)
